# baseline (speedup 1.0000x reference)
import jax
import jax.numpy as jnp
from jax import lax
from jax.experimental import pallas as pl
from jax.experimental.pallas import tpu as pltpu
from jax.experimental.pallas import tpu_sc as plsc

B, T, D = 16, 4096, 1024
NC, NS, L = 2, 16, 16
BPC = B // NC
CH = 48
NSL = D // L


def _extract(vec, k):
    lanes = lax.iota(jnp.int32, L)
    return jnp.max(jnp.where(lanes == k, vec, 0))


def _body(x_hbm, n_hbm, out_hbm, nvec, buf, acc, spacc, spfin, sem0, sem1):
    c = lax.axis_index("c")
    s = lax.axis_index("s")
    lanes = lax.iota(jnp.int32, L)
    zf = jnp.zeros((L,), jnp.float32)

    pltpu.sync_copy(n_hbm, nvec)
    nv = nvec[...]
    in_core = (lanes >= c * BPC) & (lanes < (c + 1) * BPC)
    sel = jnp.where(in_core, nv, 0)
    pre = plsc.cumsum(sel)
    excl = pre - sel
    total = _extract(pre, (c + 1) * BPC - 1)

    r0 = lax.div(s * total, NS)
    r1 = lax.div((s + 1) * total, NS)

    def zbody(i, carry):
        row = lax.div(i, NSL)
        col = lax.rem(i, NSL) * L
        acc[row, pl.ds(col, L)] = zf
        return carry

    lax.fori_loop(0, BPC * NSL, zbody, jnp.int32(0))

    def _wait(parity):
        @pl.when(parity == 0)
        def _():
            pltpu.make_async_copy(x_hbm.at[pl.ds(0, CH)],
                                  buf.at[pl.ds(0, CH)], sem0).wait()

        @pl.when(parity == 1)
        def _():
            pltpu.make_async_copy(x_hbm.at[pl.ds(0, CH)],
                                  buf.at[pl.ds(CH, CH)], sem1).wait()

    def _start(parity, src_off):
        @pl.when(parity == 0)
        def _():
            pltpu.async_copy(x_hbm.at[pl.ds(src_off, CH)],
                             buf.at[pl.ds(0, CH)], sem0)

        @pl.when(parity == 1)
        def _():
            pltpu.async_copy(x_hbm.at[pl.ds(src_off, CH)],
                             buf.at[pl.ds(CH, CH)], sem1)

    for bl in range(BPC):
        b_lo = _extract(excl, c * BPC + bl)
        b_hi = _extract(pre, c * BPC + bl)
        lo = jnp.maximum(r0, b_lo)
        hi = jnp.minimum(r1, b_hi)
        t_lo = lo - b_lo
        t_hi = hi - b_lo
        n_rows = t_hi - t_lo
        base = (c * BPC + bl) * T

        @pl.when(n_rows > 0)
        def _(bl=bl, t_lo=t_lo, t_hi=t_hi, base=base):
            a0 = t_lo & jnp.int32(-8)
            span = t_hi - a0
            nch = lax.div(span + (CH - 1), CH)

            def chunk_src(k):
                w = a0 + k * CH
                st = pl.multiple_of(jnp.minimum(w, T - CH), 8)
                return w, st

            w0, st0 = chunk_src(jnp.int32(0))
            _start(jnp.int32(0), pl.multiple_of(base + st0, 8))

            def chunk(k, carry):
                parity = k & 1
                off = parity * CH
                w, st = chunk_src(k)
                c_lo = jnp.maximum(t_lo, w)
                c_hi = jnp.minimum(t_hi, w + CH)
                _wait(parity)

                @pl.when(k + 1 < nch)
                def _():
                    _, st_n = chunk_src(k + 1)
                    _start(1 - parity, pl.multiple_of(base + st_n, 8))

                def zrow(r, zcarry):
                    for ddz in range(NSL):
                        buf[r, pl.ds(ddz * L, L)] = zf
                    return zcarry

                lax.fori_loop(off, off + (c_lo - st), zrow, jnp.int32(0))
                lax.fori_loop(off + (c_hi - st), off + CH, zrow,
                              jnp.int32(0))

                def dbody(dd, dcarry):
                    sl = pl.ds(dd * L, L)
                    chains = [zf, zf, zf, zf]
                    for t in range(CH):
                        chains[t % 4] = chains[t % 4] + buf[off + t, sl]
                    acc[bl, sl] = acc[bl, sl] + (
                        (chains[0] + chains[1]) + (chains[2] + chains[3]))
                    return dcarry

                lax.fori_loop(0, NSL, dbody, jnp.int32(0))
                return carry

            lax.fori_loop(0, nch, chunk, jnp.int32(0))

    for bl in range(BPC):
        pltpu.sync_copy(acc.at[pl.ds(bl, 1)],
                        spacc.at[pl.ds(bl * NS + s, 1)])
    plsc.subcore_barrier()

    @pl.when(s < BPC)
    def _():
        pltpu.sync_copy(spacc.at[pl.ds(s * NS, NS)], acc)
        nb = _extract(nv, c * BPC + s)
        nf = nb.astype(jnp.float32)
        inv = lax.bitcast_convert_type(
            jnp.int32(0x7EF311C3) - lax.bitcast_convert_type(nf, jnp.int32),
            jnp.float32)
        for _ in range(3):
            inv = inv * (2.0 - nf * inv)

        def fbody(dd, carry):
            sl = pl.ds(dd * L, L)
            chains = [zf, zf, zf, zf]
            for w in range(NS):
                chains[w % 4] = chains[w % 4] + acc[w, sl]
            acc[0, sl] = ((chains[0] + chains[1])
                          + (chains[2] + chains[3])) * inv
            return carry

        lax.fori_loop(0, NSL, fbody, jnp.int32(0))
        pltpu.sync_copy(acc.at[pl.ds(0, 1)], spfin.at[pl.ds(s, 1)])

    plsc.subcore_barrier()

    @pl.when(s == 0)
    def _():
        pltpu.sync_copy(spfin,
                        out_hbm.at[pl.ds(pl.multiple_of(c * BPC, 8), BPC)])


@jax.jit
def kernel(x, N):
    x2 = x.reshape(B * T, D)
    n32 = N.astype(jnp.int32)
    mesh = plsc.VectorSubcoreMesh(core_axis_name="c", subcore_axis_name="s")
    f = pl.kernel(
        _body,
        out_type=jax.ShapeDtypeStruct((B, D), jnp.float32),
        mesh=mesh,
        compiler_params=pltpu.CompilerParams(needs_layout_passes=False),
        scratch_types=[
            pltpu.VMEM((L,), jnp.int32),
            pltpu.VMEM((2 * CH, D), jnp.float32),
            pltpu.VMEM((NS, D), jnp.float32),
            pltpu.VMEM_SHARED((BPC * NS, D), jnp.float32),
            pltpu.VMEM_SHARED((BPC, D), jnp.float32),
            pltpu.SemaphoreType.DMA,
            pltpu.SemaphoreType.DMA,
        ],
    )
    return f(x2, n32)


# speedup vs baseline: 1.1213x; 1.1213x over previous
import jax
import jax.numpy as jnp
from jax import lax
from jax.experimental import pallas as pl
from jax.experimental.pallas import tpu as pltpu
from jax.experimental.pallas import tpu_sc as plsc

B, T, D = 16, 4096, 1024
TCD = 512
SCD = D - TCD
BT = 512
NBLK = T // BT

NC, NS, L = 2, 16, 16
BPC = B // NC
CH = 96
NSL = SCD // L


def _extract(vec, k):
    lanes = lax.iota(jnp.int32, L)
    return jnp.max(jnp.where(lanes == k, vec, 0))


def _tc_body(kblk_ref, n_ref, x_ref, o_ref):
    b = pl.program_id(0)
    j = pl.program_id(1)

    @pl.when(j == 0)
    def _():
        o_ref[...] = jnp.zeros_like(o_ref)

    kb = kblk_ref[b]

    @pl.when(j < kb)
    def _():
        rem = n_ref[b] - j * BT
        tvec = lax.broadcasted_iota(jnp.int32, (BT, 1), 0)
        xb = jnp.where(tvec < rem, x_ref[...], 0.0)
        o_ref[...] += jnp.sum(xb, axis=0, keepdims=True)

    @pl.when(j == NBLK - 1)
    def _():
        o_ref[...] = o_ref[...] / n_ref[b].astype(jnp.float32)


def _tc_call(x, n32):
    kblk = lax.div(n32 + (BT - 1), BT)
    grid_spec = pltpu.PrefetchScalarGridSpec(
        num_scalar_prefetch=2,
        grid=(B, NBLK),
        in_specs=[
            pl.BlockSpec((None, BT, TCD),
                         lambda b, j, kblk, nn: (b, jnp.minimum(j, kblk[b] - 1), 0)),
        ],
        out_specs=pl.BlockSpec((None, 1, TCD), lambda b, j, kblk, nn: (b, 0, 0)),
    )
    out = pl.pallas_call(
        _tc_body,
        grid_spec=grid_spec,
        out_shape=jax.ShapeDtypeStruct((B, 1, TCD), jnp.float32),
        compiler_params=pltpu.CompilerParams(
            dimension_semantics=("arbitrary", "arbitrary")),
    )(kblk, n32, x)
    return out.reshape(B, TCD)


def _sc_body(x_hbm, n_hbm, out_hbm, nvec, buf, acc, spacc, spfin, sem0, sem1):
    c = lax.axis_index("c")
    s = lax.axis_index("s")
    lanes = lax.iota(jnp.int32, L)
    zf = jnp.zeros((L,), jnp.float32)

    pltpu.sync_copy(n_hbm, nvec)
    nv = nvec[...]
    in_core = (lanes >= c * BPC) & (lanes < (c + 1) * BPC)
    sel = jnp.where(in_core, nv, 0)
    pre = plsc.cumsum(sel)
    excl = pre - sel
    total = _extract(pre, (c + 1) * BPC - 1)

    r0 = lax.div(s * total, NS)
    r1 = lax.div((s + 1) * total, NS)

    def zbody(i, carry):
        row = lax.div(i, NSL)
        col = lax.rem(i, NSL) * L
        acc[row, pl.ds(col, L)] = zf
        return carry

    lax.fori_loop(0, BPC * NSL, zbody, jnp.int32(0))

    def _wait(parity):
        @pl.when(parity == 0)
        def _():
            pltpu.make_async_copy(x_hbm.at[pl.ds(0, CH), pl.ds(TCD, SCD)],
                                  buf.at[pl.ds(0, CH)], sem0).wait()

        @pl.when(parity == 1)
        def _():
            pltpu.make_async_copy(x_hbm.at[pl.ds(0, CH), pl.ds(TCD, SCD)],
                                  buf.at[pl.ds(CH, CH)], sem1).wait()

    def _start(parity, src_off):
        @pl.when(parity == 0)
        def _():
            pltpu.async_copy(x_hbm.at[pl.ds(src_off, CH), pl.ds(TCD, SCD)],
                             buf.at[pl.ds(0, CH)], sem0)

        @pl.when(parity == 1)
        def _():
            pltpu.async_copy(x_hbm.at[pl.ds(src_off, CH), pl.ds(TCD, SCD)],
                             buf.at[pl.ds(CH, CH)], sem1)

    for bl in range(BPC):
        b_lo = _extract(excl, c * BPC + bl)
        b_hi = _extract(pre, c * BPC + bl)
        lo = jnp.maximum(r0, b_lo)
        hi = jnp.minimum(r1, b_hi)
        t_lo = lo - b_lo
        t_hi = hi - b_lo
        n_rows = t_hi - t_lo
        base = (c * BPC + bl) * T

        @pl.when(n_rows > 0)
        def _(bl=bl, t_lo=t_lo, t_hi=t_hi, base=base):
            a0 = t_lo & jnp.int32(-8)
            span = t_hi - a0
            nch = lax.div(span + (CH - 1), CH)

            def chunk_src(k):
                w = a0 + k * CH
                st = pl.multiple_of(jnp.minimum(w, T - CH), 8)
                return w, st

            w0, st0 = chunk_src(jnp.int32(0))
            _start(jnp.int32(0), pl.multiple_of(base + st0, 8))

            def chunk(k, carry):
                parity = k & 1
                off = parity * CH
                w, st = chunk_src(k)
                c_lo = jnp.maximum(t_lo, w)
                c_hi = jnp.minimum(t_hi, w + CH)
                _wait(parity)

                @pl.when(k + 1 < nch)
                def _():
                    _, st_n = chunk_src(k + 1)
                    _start(1 - parity, pl.multiple_of(base + st_n, 8))

                def zrow(r, zcarry):
                    for ddz in range(NSL):
                        buf[r, pl.ds(ddz * L, L)] = zf
                    return zcarry

                lax.fori_loop(off, off + (c_lo - st), zrow, jnp.int32(0))
                lax.fori_loop(off + (c_hi - st), off + CH, zrow,
                              jnp.int32(0))

                def dbody(dd, dcarry):
                    sl = pl.ds(dd * L, L)
                    chains = [zf, zf, zf, zf]
                    for t in range(CH):
                        chains[t % 4] = chains[t % 4] + buf[off + t, sl]
                    acc[bl, sl] = acc[bl, sl] + (
                        (chains[0] + chains[1]) + (chains[2] + chains[3]))
                    return dcarry

                lax.fori_loop(0, NSL, dbody, jnp.int32(0))
                return carry

            lax.fori_loop(0, nch, chunk, jnp.int32(0))

    for bl in range(BPC):
        pltpu.sync_copy(acc.at[pl.ds(bl, 1)],
                        spacc.at[pl.ds(bl * NS + s, 1)])
    plsc.subcore_barrier()

    @pl.when(s < BPC)
    def _():
        pltpu.sync_copy(spacc.at[pl.ds(s * NS, NS)], acc)
        nb = _extract(nv, c * BPC + s)
        nf = nb.astype(jnp.float32)
        inv = lax.bitcast_convert_type(
            jnp.int32(0x7EF311C3) - lax.bitcast_convert_type(nf, jnp.int32),
            jnp.float32)
        for _ in range(3):
            inv = inv * (2.0 - nf * inv)

        def fbody(dd, carry):
            sl = pl.ds(dd * L, L)
            chains = [zf, zf, zf, zf]
            for w in range(NS):
                chains[w % 4] = chains[w % 4] + acc[w, sl]
            acc[0, sl] = ((chains[0] + chains[1])
                          + (chains[2] + chains[3])) * inv
            return carry

        lax.fori_loop(0, NSL, fbody, jnp.int32(0))
        pltpu.sync_copy(acc.at[pl.ds(0, 1)], spfin.at[pl.ds(s, 1)])

    plsc.subcore_barrier()

    @pl.when(s == 0)
    def _():
        pltpu.sync_copy(spfin,
                        out_hbm.at[pl.ds(pl.multiple_of(c * BPC, 8), BPC)])


def _sc_call(x2, n32):
    mesh = plsc.VectorSubcoreMesh(core_axis_name="c", subcore_axis_name="s")
    f = pl.kernel(
        _sc_body,
        out_type=jax.ShapeDtypeStruct((B, SCD), jnp.float32),
        mesh=mesh,
        compiler_params=pltpu.CompilerParams(needs_layout_passes=False),
        scratch_types=[
            pltpu.VMEM((L,), jnp.int32),
            pltpu.VMEM((2 * CH, SCD), jnp.float32),
            pltpu.VMEM((NS, SCD), jnp.float32),
            pltpu.VMEM_SHARED((BPC * NS, SCD), jnp.float32),
            pltpu.VMEM_SHARED((BPC, SCD), jnp.float32),
            pltpu.SemaphoreType.DMA,
            pltpu.SemaphoreType.DMA,
        ],
    )
    return f(x2, n32)


@jax.jit
def kernel(x, N):
    n32 = N.astype(jnp.int32)
    sc_out = _sc_call(x.reshape(B * T, D), n32)
    tc_out = _tc_call(x, n32)
    return jnp.concatenate([tc_out, sc_out], axis=1)


# speedup vs baseline: 1.2293x; 1.0963x over previous
import jax
import jax.numpy as jnp
from jax import lax
from jax.experimental import pallas as pl
from jax.experimental.pallas import tpu as pltpu
from jax.experimental.pallas import tpu_sc as plsc

B, T, D = 16, 4096, 1024
TB = 11
SB = B - TB
H0 = (SB + 1) // 2
BT = 1024
NBLK = T // BT

NC, NS, L = 2, 16, 16
CH = 48
NSL = D // L


def _extract(vec, k):
    lanes = lax.iota(jnp.int32, L)
    return jnp.max(jnp.where(lanes == k, vec, 0))


def _tc_body(kblk_ref, n_ref, x_ref, o_ref):
    b = pl.program_id(0)
    j = pl.program_id(1)

    @pl.when(j == 0)
    def _():
        o_ref[...] = jnp.zeros_like(o_ref)

    kb = kblk_ref[b]

    @pl.when(j < kb)
    def _():
        rem = n_ref[b] - j * BT
        tvec = lax.broadcasted_iota(jnp.int32, (BT, 1), 0)
        xb = jnp.where(tvec < rem, x_ref[...], 0.0)
        o_ref[...] += jnp.sum(xb, axis=0, keepdims=True)

    @pl.when(j == NBLK - 1)
    def _():
        o_ref[...] = o_ref[...] / n_ref[b].astype(jnp.float32)


def _tc_call(x, n32):
    kblk = lax.div(n32 + (BT - 1), BT)
    grid_spec = pltpu.PrefetchScalarGridSpec(
        num_scalar_prefetch=2,
        grid=(TB, NBLK),
        in_specs=[
            pl.BlockSpec((None, BT, D),
                         lambda b, j, kblk, nn: (b, jnp.minimum(j, kblk[b] - 1), 0)),
        ],
        out_specs=pl.BlockSpec((None, 1, D), lambda b, j, kblk, nn: (b, 0, 0)),
    )
    out = pl.pallas_call(
        _tc_body,
        grid_spec=grid_spec,
        out_shape=jax.ShapeDtypeStruct((TB, 1, D), jnp.float32),
        compiler_params=pltpu.CompilerParams(
            dimension_semantics=("arbitrary", "arbitrary")),
    )(kblk, n32, x)
    return out.reshape(TB, D)


def _sc_body(x_hbm, n_hbm, out_hbm, nvec, buf, acc, spacc, spfin, sem0, sem1):
    c = lax.axis_index("c")
    s = lax.axis_index("s")
    lanes = lax.iota(jnp.int32, L)
    zf = jnp.zeros((L,), jnp.float32)

    start_b = jnp.where(c == 0, TB, TB + H0)
    cnt = jnp.where(c == 0, H0, SB - H0)

    pltpu.sync_copy(n_hbm, nvec)
    nv = nvec[...]
    in_core = (lanes >= start_b) & (lanes < start_b + cnt)
    sel = jnp.where(in_core, nv, 0)
    pre = plsc.cumsum(sel)
    excl = pre - sel
    total = _extract(pre, start_b + cnt - 1)

    r0 = lax.div(s * total, NS)
    r1 = lax.div((s + 1) * total, NS)

    def zbody(i, carry):
        row = lax.div(i, NSL)
        col = lax.rem(i, NSL) * L
        acc[row, pl.ds(col, L)] = zf
        return carry

    lax.fori_loop(0, H0 * NSL, zbody, jnp.int32(0))

    def _wait(parity):
        @pl.when(parity == 0)
        def _():
            pltpu.make_async_copy(x_hbm.at[pl.ds(0, CH)],
                                  buf.at[pl.ds(0, CH)], sem0).wait()

        @pl.when(parity == 1)
        def _():
            pltpu.make_async_copy(x_hbm.at[pl.ds(0, CH)],
                                  buf.at[pl.ds(CH, CH)], sem1).wait()

    def _start(parity, src_off):
        @pl.when(parity == 0)
        def _():
            pltpu.async_copy(x_hbm.at[pl.ds(src_off, CH)],
                             buf.at[pl.ds(0, CH)], sem0)

        @pl.when(parity == 1)
        def _():
            pltpu.async_copy(x_hbm.at[pl.ds(src_off, CH)],
                             buf.at[pl.ds(CH, CH)], sem1)

    for bl in range(H0):
        b_lo = _extract(excl, start_b + bl)
        b_hi = _extract(pre, start_b + bl)
        lo = jnp.maximum(r0, b_lo)
        hi = jnp.minimum(r1, b_hi)
        t_lo = lo - b_lo
        t_hi = hi - b_lo
        n_rows = t_hi - t_lo
        base = (start_b + bl) * T

        @pl.when(n_rows > 0)
        def _(bl=bl, t_lo=t_lo, t_hi=t_hi, base=base):
            a0 = t_lo & jnp.int32(-8)
            span = t_hi - a0
            nch = lax.div(span + (CH - 1), CH)

            def chunk_src(k):
                w = a0 + k * CH
                st = pl.multiple_of(jnp.minimum(w, T - CH), 8)
                return w, st

            w0, st0 = chunk_src(jnp.int32(0))
            _start(jnp.int32(0), pl.multiple_of(base + st0, 8))

            def chunk(k, carry):
                parity = k & 1
                off = parity * CH
                w, st = chunk_src(k)
                c_lo = jnp.maximum(t_lo, w)
                c_hi = jnp.minimum(t_hi, w + CH)
                _wait(parity)

                @pl.when(k + 1 < nch)
                def _():
                    _, st_n = chunk_src(k + 1)
                    _start(1 - parity, pl.multiple_of(base + st_n, 8))

                def zrow(r, zcarry):
                    for ddz in range(NSL):
                        buf[r, pl.ds(ddz * L, L)] = zf
                    return zcarry

                lax.fori_loop(off, off + (c_lo - st), zrow, jnp.int32(0))
                lax.fori_loop(off + (c_hi - st), off + CH, zrow,
                              jnp.int32(0))

                def dbody(dd, dcarry):
                    sl = pl.ds(dd * L, L)
                    chains = [zf, zf, zf, zf]
                    for t in range(CH):
                        chains[t % 4] = chains[t % 4] + buf[off + t, sl]
                    acc[bl, sl] = acc[bl, sl] + (
                        (chains[0] + chains[1]) + (chains[2] + chains[3]))
                    return dcarry

                lax.fori_loop(0, NSL, dbody, jnp.int32(0))
                return carry

            lax.fori_loop(0, nch, chunk, jnp.int32(0))

    for bl in range(H0):
        pltpu.sync_copy(acc.at[pl.ds(bl, 1)],
                        spacc.at[pl.ds(bl * NS + s, 1)])
    plsc.subcore_barrier()

    @pl.when(s < cnt)
    def _():
        pltpu.sync_copy(spacc.at[pl.ds(s * NS, NS)], acc)
        nb = _extract(nv, start_b + s)
        nf = nb.astype(jnp.float32)
        inv = lax.bitcast_convert_type(
            jnp.int32(0x7EF311C3) - lax.bitcast_convert_type(nf, jnp.int32),
            jnp.float32)
        for _ in range(3):
            inv = inv * (2.0 - nf * inv)

        def fbody(dd, carry):
            sl = pl.ds(dd * L, L)
            chains = [zf, zf, zf, zf]
            for w in range(NS):
                chains[w % 4] = chains[w % 4] + acc[w, sl]
            acc[0, sl] = ((chains[0] + chains[1])
                          + (chains[2] + chains[3])) * inv
            return carry

        lax.fori_loop(0, NSL, fbody, jnp.int32(0))
        pltpu.sync_copy(acc.at[pl.ds(0, 1)], spfin.at[pl.ds(s, 1)])

    plsc.subcore_barrier()

    @pl.when(s == 0)
    def _():
        pltpu.sync_copy(spfin,
                        out_hbm.at[pl.ds(pl.multiple_of(c * 8, 8), 8)])


def _sc_call(x2, n32):
    mesh = plsc.VectorSubcoreMesh(core_axis_name="c", subcore_axis_name="s")
    f = pl.kernel(
        _sc_body,
        out_type=jax.ShapeDtypeStruct((2 * 8, D), jnp.float32),
        mesh=mesh,
        compiler_params=pltpu.CompilerParams(needs_layout_passes=False),
        scratch_types=[
            pltpu.VMEM((L,), jnp.int32),
            pltpu.VMEM((2 * CH, D), jnp.float32),
            pltpu.VMEM((NS, D), jnp.float32),
            pltpu.VMEM_SHARED((H0 * NS, D), jnp.float32),
            pltpu.VMEM_SHARED((8, D), jnp.float32),
            pltpu.SemaphoreType.DMA,
            pltpu.SemaphoreType.DMA,
        ],
    )
    return f(x2, n32)


@jax.jit
def kernel(x, N):
    n32 = N.astype(jnp.int32)
    sc_out = _sc_call(x.reshape(B * T, D), n32)
    tc_out = _tc_call(x, n32)
    return jnp.concatenate(
        [tc_out, sc_out[0:H0], sc_out[8:8 + (SB - H0)]], axis=0)


# speedup vs baseline: 1.2939x; 1.0526x over previous
import jax
import jax.numpy as jnp
from jax import lax
from jax.experimental import pallas as pl
from jax.experimental.pallas import tpu as pltpu
from jax.experimental.pallas import tpu_sc as plsc

B, T, D = 16, 4096, 1024
TB = 10
SB = B - TB
H0 = (SB + 1) // 2
BT = 1024
NBLK = T // BT

NC, NS, L = 2, 16, 16
CH = 48
NSL = D // L


def _extract(vec, k):
    lanes = lax.iota(jnp.int32, L)
    return jnp.max(jnp.where(lanes == k, vec, 0))


def _tc_body(kblk_ref, n_ref, x_ref, o_ref):
    b = pl.program_id(0)
    j = pl.program_id(1)

    @pl.when(j == 0)
    def _():
        o_ref[...] = jnp.zeros_like(o_ref)

    kb = kblk_ref[b]

    @pl.when(j < kb)
    def _():
        rem = n_ref[b] - j * BT
        tvec = lax.broadcasted_iota(jnp.int32, (BT, 1), 0)
        xb = jnp.where(tvec < rem, x_ref[...], 0.0)
        o_ref[...] += jnp.sum(xb, axis=0, keepdims=True)

    @pl.when(j == NBLK - 1)
    def _():
        o_ref[...] = o_ref[...] / n_ref[b].astype(jnp.float32)


def _tc_call(x, n32):
    kblk = lax.div(n32 + (BT - 1), BT)
    grid_spec = pltpu.PrefetchScalarGridSpec(
        num_scalar_prefetch=2,
        grid=(TB, NBLK),
        in_specs=[
            pl.BlockSpec((None, BT, D),
                         lambda b, j, kblk, nn: (b, jnp.minimum(j, kblk[b] - 1), 0)),
        ],
        out_specs=pl.BlockSpec((None, 1, D), lambda b, j, kblk, nn: (b, 0, 0)),
    )
    out = pl.pallas_call(
        _tc_body,
        grid_spec=grid_spec,
        out_shape=jax.ShapeDtypeStruct((TB, 1, D), jnp.float32),
        compiler_params=pltpu.CompilerParams(
            dimension_semantics=("arbitrary", "arbitrary")),
    )(kblk, n32, x)
    return out.reshape(TB, D)


def _sc_body(x_hbm, n_hbm, out_hbm, nvec, buf, acc, spacc, spfin, sem0, sem1):
    c = lax.axis_index("c")
    s = lax.axis_index("s")
    lanes = lax.iota(jnp.int32, L)
    zf = jnp.zeros((L,), jnp.float32)

    start_b = jnp.where(c == 0, TB, TB + H0)
    cnt = jnp.where(c == 0, H0, SB - H0)

    pltpu.sync_copy(n_hbm, nvec)
    nv = nvec[...]
    in_core = (lanes >= start_b) & (lanes < start_b + cnt)
    sel = jnp.where(in_core, nv, 0)
    pre = plsc.cumsum(sel)
    excl = pre - sel
    total = _extract(pre, start_b + cnt - 1)

    r0 = lax.div(s * total, NS)
    r1 = lax.div((s + 1) * total, NS)

    def zbody(i, carry):
        row = lax.div(i, NSL)
        col = lax.rem(i, NSL) * L
        acc[row, pl.ds(col, L)] = zf
        return carry

    lax.fori_loop(0, H0 * NSL, zbody, jnp.int32(0))

    def _wait(parity):
        @pl.when(parity == 0)
        def _():
            pltpu.make_async_copy(x_hbm.at[pl.ds(0, CH)],
                                  buf.at[pl.ds(0, CH)], sem0).wait()

        @pl.when(parity == 1)
        def _():
            pltpu.make_async_copy(x_hbm.at[pl.ds(0, CH)],
                                  buf.at[pl.ds(CH, CH)], sem1).wait()

    def _start(parity, src_off):
        @pl.when(parity == 0)
        def _():
            pltpu.async_copy(x_hbm.at[pl.ds(src_off, CH)],
                             buf.at[pl.ds(0, CH)], sem0)

        @pl.when(parity == 1)
        def _():
            pltpu.async_copy(x_hbm.at[pl.ds(src_off, CH)],
                             buf.at[pl.ds(CH, CH)], sem1)

    for bl in range(H0):
        b_lo = _extract(excl, start_b + bl)
        b_hi = _extract(pre, start_b + bl)
        lo = jnp.maximum(r0, b_lo)
        hi = jnp.minimum(r1, b_hi)
        t_lo = lo - b_lo
        t_hi = hi - b_lo
        n_rows = t_hi - t_lo
        base = (start_b + bl) * T

        @pl.when(n_rows > 0)
        def _(bl=bl, t_lo=t_lo, t_hi=t_hi, base=base):
            a0 = t_lo & jnp.int32(-8)
            span = t_hi - a0
            nch = lax.div(span + (CH - 1), CH)

            def chunk_src(k):
                w = a0 + k * CH
                st = pl.multiple_of(jnp.minimum(w, T - CH), 8)
                return w, st

            w0, st0 = chunk_src(jnp.int32(0))
            _start(jnp.int32(0), pl.multiple_of(base + st0, 8))

            def chunk(k, carry):
                parity = k & 1
                off = parity * CH
                w, st = chunk_src(k)
                c_lo = jnp.maximum(t_lo, w)
                c_hi = jnp.minimum(t_hi, w + CH)
                _wait(parity)

                @pl.when(k + 1 < nch)
                def _():
                    _, st_n = chunk_src(k + 1)
                    _start(1 - parity, pl.multiple_of(base + st_n, 8))

                def zrow(r, zcarry):
                    for ddz in range(NSL):
                        buf[r, pl.ds(ddz * L, L)] = zf
                    return zcarry

                lax.fori_loop(off, off + (c_lo - st), zrow, jnp.int32(0))
                lax.fori_loop(off + (c_hi - st), off + CH, zrow,
                              jnp.int32(0))

                def dbody(dd, dcarry):
                    sl = pl.ds(dd * L, L)
                    chains = [zf, zf, zf, zf]
                    for t in range(CH):
                        chains[t % 4] = chains[t % 4] + buf[off + t, sl]
                    acc[bl, sl] = acc[bl, sl] + (
                        (chains[0] + chains[1]) + (chains[2] + chains[3]))
                    return dcarry

                lax.fori_loop(0, NSL, dbody, jnp.int32(0))
                return carry

            lax.fori_loop(0, nch, chunk, jnp.int32(0))

    for bl in range(H0):
        pltpu.sync_copy(acc.at[pl.ds(bl, 1)],
                        spacc.at[pl.ds(bl * NS + s, 1)])
    plsc.subcore_barrier()

    @pl.when(s < cnt)
    def _():
        pltpu.sync_copy(spacc.at[pl.ds(s * NS, NS)], acc)
        nb = _extract(nv, start_b + s)
        nf = nb.astype(jnp.float32)
        inv = lax.bitcast_convert_type(
            jnp.int32(0x7EF311C3) - lax.bitcast_convert_type(nf, jnp.int32),
            jnp.float32)
        for _ in range(3):
            inv = inv * (2.0 - nf * inv)

        def fbody(dd, carry):
            sl = pl.ds(dd * L, L)
            chains = [zf, zf, zf, zf]
            for w in range(NS):
                chains[w % 4] = chains[w % 4] + acc[w, sl]
            acc[0, sl] = ((chains[0] + chains[1])
                          + (chains[2] + chains[3])) * inv
            return carry

        lax.fori_loop(0, NSL, fbody, jnp.int32(0))
        pltpu.sync_copy(acc.at[pl.ds(0, 1)], spfin.at[pl.ds(s, 1)])

    plsc.subcore_barrier()

    @pl.when(s == 0)
    def _():
        pltpu.sync_copy(spfin,
                        out_hbm.at[pl.ds(pl.multiple_of(c * 8, 8), 8)])


def _sc_call(x2, n32):
    mesh = plsc.VectorSubcoreMesh(core_axis_name="c", subcore_axis_name="s")
    f = pl.kernel(
        _sc_body,
        out_type=jax.ShapeDtypeStruct((2 * 8, D), jnp.float32),
        mesh=mesh,
        compiler_params=pltpu.CompilerParams(needs_layout_passes=False),
        scratch_types=[
            pltpu.VMEM((L,), jnp.int32),
            pltpu.VMEM((2 * CH, D), jnp.float32),
            pltpu.VMEM((NS, D), jnp.float32),
            pltpu.VMEM_SHARED((H0 * NS, D), jnp.float32),
            pltpu.VMEM_SHARED((8, D), jnp.float32),
            pltpu.SemaphoreType.DMA,
            pltpu.SemaphoreType.DMA,
        ],
    )
    return f(x2, n32)


@jax.jit
def kernel(x, N):
    n32 = N.astype(jnp.int32)
    sc_out = _sc_call(x.reshape(B * T, D), n32)
    tc_out = _tc_call(x, n32)
    return jnp.concatenate(
        [tc_out, sc_out[0:H0], sc_out[8:8 + (SB - H0)]], axis=0)


# speedup vs baseline: 1.3134x; 1.0151x over previous
import jax
import jax.numpy as jnp
from jax import lax
from jax.experimental import pallas as pl
from jax.experimental.pallas import tpu as pltpu
from jax.experimental.pallas import tpu_sc as plsc

B, T, D = 16, 4096, 1024
TB = 9
SB = B - TB
H0 = (SB + 1) // 2
BT = 1024
NBLK = T // BT

NC, NS, L = 2, 16, 16
CH = 48
NSL = D // L


def _extract(vec, k):
    lanes = lax.iota(jnp.int32, L)
    return jnp.max(jnp.where(lanes == k, vec, 0))


def _tc_body(kblk_ref, n_ref, x_ref, o_ref):
    b = pl.program_id(0)
    j = pl.program_id(1)

    @pl.when(j == 0)
    def _():
        o_ref[...] = jnp.zeros_like(o_ref)

    kb = kblk_ref[b]

    @pl.when(j < kb)
    def _():
        rem = n_ref[b] - j * BT
        tvec = lax.broadcasted_iota(jnp.int32, (BT, 1), 0)
        xb = jnp.where(tvec < rem, x_ref[...], 0.0)
        o_ref[...] += jnp.sum(xb, axis=0, keepdims=True)

    @pl.when(j == NBLK - 1)
    def _():
        o_ref[...] = o_ref[...] / n_ref[b].astype(jnp.float32)


def _tc_call(x, n32):
    kblk = lax.div(n32 + (BT - 1), BT)
    grid_spec = pltpu.PrefetchScalarGridSpec(
        num_scalar_prefetch=2,
        grid=(TB, NBLK),
        in_specs=[
            pl.BlockSpec((None, BT, D),
                         lambda b, j, kblk, nn: (b, jnp.minimum(j, kblk[b] - 1), 0)),
        ],
        out_specs=pl.BlockSpec((None, 1, D), lambda b, j, kblk, nn: (b, 0, 0)),
    )
    out = pl.pallas_call(
        _tc_body,
        grid_spec=grid_spec,
        out_shape=jax.ShapeDtypeStruct((TB, 1, D), jnp.float32),
        compiler_params=pltpu.CompilerParams(
            dimension_semantics=("arbitrary", "arbitrary")),
    )(kblk, n32, x)
    return out.reshape(TB, D)


def _sc_body(x_hbm, n_hbm, out_hbm, nvec, buf, acc, spacc, spfin, sem0, sem1):
    c = lax.axis_index("c")
    s = lax.axis_index("s")
    lanes = lax.iota(jnp.int32, L)
    zf = jnp.zeros((L,), jnp.float32)

    start_b = jnp.where(c == 0, TB, TB + H0)
    cnt = jnp.where(c == 0, H0, SB - H0)

    pltpu.sync_copy(n_hbm, nvec)
    nv = nvec[...]
    in_core = (lanes >= start_b) & (lanes < start_b + cnt)
    sel = jnp.where(in_core, nv, 0)
    pre = plsc.cumsum(sel)
    excl = pre - sel
    total = _extract(pre, start_b + cnt - 1)

    r0 = lax.div(s * total, NS)
    r1 = lax.div((s + 1) * total, NS)

    def zbody(i, carry):
        row = lax.div(i, NSL)
        col = lax.rem(i, NSL) * L
        acc[row, pl.ds(col, L)] = zf
        return carry

    lax.fori_loop(0, H0 * NSL, zbody, jnp.int32(0))

    def _wait(parity):
        @pl.when(parity == 0)
        def _():
            pltpu.make_async_copy(x_hbm.at[pl.ds(0, CH)],
                                  buf.at[pl.ds(0, CH)], sem0).wait()

        @pl.when(parity == 1)
        def _():
            pltpu.make_async_copy(x_hbm.at[pl.ds(0, CH)],
                                  buf.at[pl.ds(CH, CH)], sem1).wait()

    def _start(parity, src_off):
        @pl.when(parity == 0)
        def _():
            pltpu.async_copy(x_hbm.at[pl.ds(src_off, CH)],
                             buf.at[pl.ds(0, CH)], sem0)

        @pl.when(parity == 1)
        def _():
            pltpu.async_copy(x_hbm.at[pl.ds(src_off, CH)],
                             buf.at[pl.ds(CH, CH)], sem1)

    for bl in range(H0):
        b_lo = _extract(excl, start_b + bl)
        b_hi = _extract(pre, start_b + bl)
        lo = jnp.maximum(r0, b_lo)
        hi = jnp.minimum(r1, b_hi)
        t_lo = lo - b_lo
        t_hi = hi - b_lo
        n_rows = t_hi - t_lo
        base = (start_b + bl) * T

        @pl.when(n_rows > 0)
        def _(bl=bl, t_lo=t_lo, t_hi=t_hi, base=base):
            a0 = t_lo & jnp.int32(-8)
            span = t_hi - a0
            nch = lax.div(span + (CH - 1), CH)

            def chunk_src(k):
                w = a0 + k * CH
                st = pl.multiple_of(jnp.minimum(w, T - CH), 8)
                return w, st

            w0, st0 = chunk_src(jnp.int32(0))
            _start(jnp.int32(0), pl.multiple_of(base + st0, 8))

            def chunk(k, carry):
                parity = k & 1
                off = parity * CH
                w, st = chunk_src(k)
                c_lo = jnp.maximum(t_lo, w)
                c_hi = jnp.minimum(t_hi, w + CH)
                _wait(parity)

                @pl.when(k + 1 < nch)
                def _():
                    _, st_n = chunk_src(k + 1)
                    _start(1 - parity, pl.multiple_of(base + st_n, 8))

                def zrow(r, zcarry):
                    for ddz in range(NSL):
                        buf[r, pl.ds(ddz * L, L)] = zf
                    return zcarry

                lax.fori_loop(off, off + (c_lo - st), zrow, jnp.int32(0))
                lax.fori_loop(off + (c_hi - st), off + CH, zrow,
                              jnp.int32(0))

                def dbody(dd, dcarry):
                    sl = pl.ds(dd * L, L)
                    chains = [zf, zf, zf, zf]
                    for t in range(CH):
                        chains[t % 4] = chains[t % 4] + buf[off + t, sl]
                    acc[bl, sl] = acc[bl, sl] + (
                        (chains[0] + chains[1]) + (chains[2] + chains[3]))
                    return dcarry

                lax.fori_loop(0, NSL, dbody, jnp.int32(0))
                return carry

            lax.fori_loop(0, nch, chunk, jnp.int32(0))

    for bl in range(H0):
        pltpu.sync_copy(acc.at[pl.ds(bl, 1)],
                        spacc.at[pl.ds(bl * NS + s, 1)])
    plsc.subcore_barrier()

    @pl.when(s < cnt)
    def _():
        pltpu.sync_copy(spacc.at[pl.ds(s * NS, NS)], acc)
        nb = _extract(nv, start_b + s)
        nf = nb.astype(jnp.float32)
        inv = lax.bitcast_convert_type(
            jnp.int32(0x7EF311C3) - lax.bitcast_convert_type(nf, jnp.int32),
            jnp.float32)
        for _ in range(3):
            inv = inv * (2.0 - nf * inv)

        def fbody(dd, carry):
            sl = pl.ds(dd * L, L)
            chains = [zf, zf, zf, zf]
            for w in range(NS):
                chains[w % 4] = chains[w % 4] + acc[w, sl]
            acc[0, sl] = ((chains[0] + chains[1])
                          + (chains[2] + chains[3])) * inv
            return carry

        lax.fori_loop(0, NSL, fbody, jnp.int32(0))
        pltpu.sync_copy(acc.at[pl.ds(0, 1)], spfin.at[pl.ds(s, 1)])

    plsc.subcore_barrier()

    @pl.when(s == 0)
    def _():
        pltpu.sync_copy(spfin,
                        out_hbm.at[pl.ds(pl.multiple_of(c * 8, 8), 8)])


def _sc_call(x2, n32):
    mesh = plsc.VectorSubcoreMesh(core_axis_name="c", subcore_axis_name="s")
    f = pl.kernel(
        _sc_body,
        out_type=jax.ShapeDtypeStruct((2 * 8, D), jnp.float32),
        mesh=mesh,
        compiler_params=pltpu.CompilerParams(needs_layout_passes=False),
        scratch_types=[
            pltpu.VMEM((L,), jnp.int32),
            pltpu.VMEM((2 * CH, D), jnp.float32),
            pltpu.VMEM((NS, D), jnp.float32),
            pltpu.VMEM_SHARED((H0 * NS, D), jnp.float32),
            pltpu.VMEM_SHARED((8, D), jnp.float32),
            pltpu.SemaphoreType.DMA,
            pltpu.SemaphoreType.DMA,
        ],
    )
    return f(x2, n32)


@jax.jit
def kernel(x, N):
    n32 = N.astype(jnp.int32)
    sc_out = _sc_call(x.reshape(B * T, D), n32)
    tc_out = _tc_call(x, n32)
    return jnp.concatenate(
        [tc_out, sc_out[0:H0], sc_out[8:8 + (SB - H0)]], axis=0)


# speedup vs baseline: 1.3665x; 1.0404x over previous
import jax
import jax.numpy as jnp
from jax import lax
from jax.experimental import pallas as pl
from jax.experimental.pallas import tpu as pltpu
from jax.experimental.pallas import tpu_sc as plsc

B, T, D = 16, 4096, 1024
TB = 8
SB = B - TB
H0 = (SB + 1) // 2
BT = 1024
NBLK = T // BT

NC, NS, L = 2, 16, 16
CH = 48
NSL = D // L


def _extract(vec, k):
    lanes = lax.iota(jnp.int32, L)
    return jnp.max(jnp.where(lanes == k, vec, 0))


def _tc_body(kblk_ref, n_ref, x_ref, o_ref):
    b = pl.program_id(0)
    j = pl.program_id(1)

    @pl.when(j == 0)
    def _():
        o_ref[...] = jnp.zeros_like(o_ref)

    kb = kblk_ref[b]

    @pl.when(j < kb)
    def _():
        rem = n_ref[b] - j * BT
        tvec = lax.broadcasted_iota(jnp.int32, (BT, 1), 0)
        xb = jnp.where(tvec < rem, x_ref[...], 0.0)
        o_ref[...] += jnp.sum(xb, axis=0, keepdims=True)

    @pl.when(j == NBLK - 1)
    def _():
        o_ref[...] = o_ref[...] / n_ref[b].astype(jnp.float32)


def _tc_call(x, n32):
    kblk = lax.div(n32 + (BT - 1), BT)
    grid_spec = pltpu.PrefetchScalarGridSpec(
        num_scalar_prefetch=2,
        grid=(TB, NBLK),
        in_specs=[
            pl.BlockSpec((None, BT, D),
                         lambda b, j, kblk, nn: (b, jnp.minimum(j, kblk[b] - 1), 0)),
        ],
        out_specs=pl.BlockSpec((None, 1, D), lambda b, j, kblk, nn: (b, 0, 0)),
    )
    out = pl.pallas_call(
        _tc_body,
        grid_spec=grid_spec,
        out_shape=jax.ShapeDtypeStruct((TB, 1, D), jnp.float32),
        compiler_params=pltpu.CompilerParams(
            dimension_semantics=("arbitrary", "arbitrary")),
    )(kblk, n32, x)
    return out.reshape(TB, D)


def _sc_body(x_hbm, n_hbm, out_hbm, nvec, buf, acc, spacc, spfin, sem0, sem1):
    c = lax.axis_index("c")
    s = lax.axis_index("s")
    lanes = lax.iota(jnp.int32, L)
    zf = jnp.zeros((L,), jnp.float32)

    start_b = jnp.where(c == 0, TB, TB + H0)
    cnt = jnp.where(c == 0, H0, SB - H0)

    pltpu.sync_copy(n_hbm, nvec)
    nv = nvec[...]
    in_core = (lanes >= start_b) & (lanes < start_b + cnt)
    sel = jnp.where(in_core, nv, 0)
    pre = plsc.cumsum(sel)
    excl = pre - sel
    total = _extract(pre, start_b + cnt - 1)

    r0 = lax.div(s * total, NS)
    r1 = lax.div((s + 1) * total, NS)

    def zbody(i, carry):
        row = lax.div(i, NSL)
        col = lax.rem(i, NSL) * L
        acc[row, pl.ds(col, L)] = zf
        return carry

    lax.fori_loop(0, H0 * NSL, zbody, jnp.int32(0))

    def _wait(parity):
        @pl.when(parity == 0)
        def _():
            pltpu.make_async_copy(x_hbm.at[pl.ds(0, CH)],
                                  buf.at[pl.ds(0, CH)], sem0).wait()

        @pl.when(parity == 1)
        def _():
            pltpu.make_async_copy(x_hbm.at[pl.ds(0, CH)],
                                  buf.at[pl.ds(CH, CH)], sem1).wait()

    def _start(parity, src_off):
        @pl.when(parity == 0)
        def _():
            pltpu.async_copy(x_hbm.at[pl.ds(src_off, CH)],
                             buf.at[pl.ds(0, CH)], sem0)

        @pl.when(parity == 1)
        def _():
            pltpu.async_copy(x_hbm.at[pl.ds(src_off, CH)],
                             buf.at[pl.ds(CH, CH)], sem1)

    for bl in range(H0):
        b_lo = _extract(excl, start_b + bl)
        b_hi = _extract(pre, start_b + bl)
        lo = jnp.maximum(r0, b_lo)
        hi = jnp.minimum(r1, b_hi)
        t_lo = lo - b_lo
        t_hi = hi - b_lo
        n_rows = t_hi - t_lo
        base = (start_b + bl) * T

        @pl.when(n_rows > 0)
        def _(bl=bl, t_lo=t_lo, t_hi=t_hi, base=base):
            a0 = t_lo & jnp.int32(-8)
            span = t_hi - a0
            nch = lax.div(span + (CH - 1), CH)

            def chunk_src(k):
                w = a0 + k * CH
                st = pl.multiple_of(jnp.minimum(w, T - CH), 8)
                return w, st

            w0, st0 = chunk_src(jnp.int32(0))
            _start(jnp.int32(0), pl.multiple_of(base + st0, 8))

            def chunk(k, carry):
                parity = k & 1
                off = parity * CH
                w, st = chunk_src(k)
                c_lo = jnp.maximum(t_lo, w)
                c_hi = jnp.minimum(t_hi, w + CH)
                _wait(parity)

                @pl.when(k + 1 < nch)
                def _():
                    _, st_n = chunk_src(k + 1)
                    _start(1 - parity, pl.multiple_of(base + st_n, 8))

                def zrow(r, zcarry):
                    for ddz in range(NSL):
                        buf[r, pl.ds(ddz * L, L)] = zf
                    return zcarry

                lax.fori_loop(off, off + (c_lo - st), zrow, jnp.int32(0))
                lax.fori_loop(off + (c_hi - st), off + CH, zrow,
                              jnp.int32(0))

                def dbody(dd, dcarry):
                    sl = pl.ds(dd * L, L)
                    chains = [zf, zf, zf, zf]
                    for t in range(CH):
                        chains[t % 4] = chains[t % 4] + buf[off + t, sl]
                    acc[bl, sl] = acc[bl, sl] + (
                        (chains[0] + chains[1]) + (chains[2] + chains[3]))
                    return dcarry

                lax.fori_loop(0, NSL, dbody, jnp.int32(0))
                return carry

            lax.fori_loop(0, nch, chunk, jnp.int32(0))

    for bl in range(H0):
        pltpu.sync_copy(acc.at[pl.ds(bl, 1)],
                        spacc.at[pl.ds(bl * NS + s, 1)])
    plsc.subcore_barrier()

    @pl.when(s < cnt)
    def _():
        pltpu.sync_copy(spacc.at[pl.ds(s * NS, NS)], acc)
        nb = _extract(nv, start_b + s)
        nf = nb.astype(jnp.float32)
        inv = lax.bitcast_convert_type(
            jnp.int32(0x7EF311C3) - lax.bitcast_convert_type(nf, jnp.int32),
            jnp.float32)
        for _ in range(3):
            inv = inv * (2.0 - nf * inv)

        def fbody(dd, carry):
            sl = pl.ds(dd * L, L)
            chains = [zf, zf, zf, zf]
            for w in range(NS):
                chains[w % 4] = chains[w % 4] + acc[w, sl]
            acc[0, sl] = ((chains[0] + chains[1])
                          + (chains[2] + chains[3])) * inv
            return carry

        lax.fori_loop(0, NSL, fbody, jnp.int32(0))
        pltpu.sync_copy(acc.at[pl.ds(0, 1)], spfin.at[pl.ds(s, 1)])

    plsc.subcore_barrier()

    @pl.when(s == 0)
    def _():
        pltpu.sync_copy(spfin,
                        out_hbm.at[pl.ds(pl.multiple_of(c * 8, 8), 8)])


def _sc_call(x2, n32):
    mesh = plsc.VectorSubcoreMesh(core_axis_name="c", subcore_axis_name="s")
    f = pl.kernel(
        _sc_body,
        out_type=jax.ShapeDtypeStruct((2 * 8, D), jnp.float32),
        mesh=mesh,
        compiler_params=pltpu.CompilerParams(needs_layout_passes=False),
        scratch_types=[
            pltpu.VMEM((L,), jnp.int32),
            pltpu.VMEM((2 * CH, D), jnp.float32),
            pltpu.VMEM((NS, D), jnp.float32),
            pltpu.VMEM_SHARED((H0 * NS, D), jnp.float32),
            pltpu.VMEM_SHARED((8, D), jnp.float32),
            pltpu.SemaphoreType.DMA,
            pltpu.SemaphoreType.DMA,
        ],
    )
    return f(x2, n32)


@jax.jit
def kernel(x, N):
    n32 = N.astype(jnp.int32)
    sc_out = _sc_call(x.reshape(B * T, D), n32)
    tc_out = _tc_call(x, n32)
    return jnp.concatenate(
        [tc_out, sc_out[0:H0], sc_out[8:8 + (SB - H0)]], axis=0)
